# baseline (speedup 1.0000x reference)
import jax
import jax.numpy as jnp
from jax import lax
from jax.experimental import pallas as pl
from jax.experimental.pallas import tpu as pltpu
from jax.experimental.pallas import tpu_sc as plsc

NUM_FUNCS = 32
TABLE_ROWS = 256
LANES = 16

_ROWS = 16384
_COLS = 100
_OUT_COLS = _COLS * NUM_FUNCS
_NW = 32
_ROWS_W = _ROWS // _NW
_CROWS = 16
_CHUNK = _CROWS * _COLS
_NCHUNK = _ROWS_W // _CROWS
_NBUF = 2


def _rbf_kernel(table_hbm, idx_hbm, out_hbm, table_v, idx_v0, idx_v1,
                out_v0, out_v1, si0, si1, so0, so1):
    wid = lax.axis_index("s") * 2 + lax.axis_index("c")
    wrow = wid * _ROWS_W
    idx_v = [idx_v0, idx_v1]
    out_v = [out_v0, out_v1]
    si = [si0, si1]
    so = [so0, so1]

    pltpu.sync_copy(table_hbm, table_v)

    lane = lax.iota(jnp.int32, LANES)

    def in_copy(b, g):
        r0 = wrow + g * _CROWS
        return pltpu.make_async_copy(
            idx_hbm.at[pl.ds(r0, _CROWS), :], idx_v[b], si[b])

    def out_copy(b, g):
        r0 = wrow + g * _CROWS
        return pltpu.make_async_copy(
            out_v[b], out_hbm.at[pl.ds(r0, _CROWS), :], so[b])

    for b in range(_NBUF):
        in_copy(b, b).start()

    def outer_body(o, carry):
        for b in range(_NBUF):
            g = o * _NBUF + b
            in_copy(b, g).wait()

            @pl.when(o > 0)
            def _wait_out():
                out_copy(b, g - _NBUF).wait()

            @plsc.parallel_loop(0, _CHUNK // (4 * LANES))
            def group_body(i):
                quads = []
                for h in range(4):
                    e = i * (4 * LANES) + h * LANES + lane
                    ri = (e * 5243) >> 19
                    c = e - ri * _COLS
                    r = plsc.load_gather(idx_v[b], [ri, c])
                    r = jnp.minimum(jnp.maximum(r, 0), TABLE_ROWS - 1)
                    quads.append((ri, c * NUM_FUNCS, r * NUM_FUNCS))

                @plsc.parallel_loop(0, NUM_FUNCS, unroll=2)
                def kbody(k):
                    jv = (lane + k) & (NUM_FUNCS - 1)
                    for ri, c32, r32 in quads:
                        vals = plsc.load_gather(table_v, [r32 + jv])
                        plsc.store_scatter(out_v[b], [ri, c32 + jv], vals)

            out_copy(b, g).start()

            @pl.when(o < _NCHUNK // _NBUF - 1)
            def _prefetch():
                in_copy(b, g + _NBUF).start()

        return carry

    lax.fori_loop(0, _NCHUNK // _NBUF, outer_body, 0)
    for b in range(_NBUF):
        out_copy(b, _NCHUNK - _NBUF + b).wait()


def kernel(tensor, int_to_feat_matrix, extra_embeddings):
    orig_shape = tensor.shape
    table = jnp.concatenate([int_to_feat_matrix, extra_embeddings], axis=0)
    table_flat = table.reshape(-1)
    idx = tensor.astype(jnp.int32)

    mesh = plsc.VectorSubcoreMesh(core_axis_name="c", subcore_axis_name="s")
    run = pl.kernel(
        _rbf_kernel,
        mesh=mesh,
        out_type=jax.ShapeDtypeStruct((_ROWS, _OUT_COLS), jnp.float32),
        scratch_types=[
            pltpu.VMEM((TABLE_ROWS * NUM_FUNCS,), jnp.float32),
            pltpu.VMEM((_CROWS, _COLS), jnp.int32),
            pltpu.VMEM((_CROWS, _COLS), jnp.int32),
            pltpu.VMEM((_CROWS, _OUT_COLS), jnp.float32),
            pltpu.VMEM((_CROWS, _OUT_COLS), jnp.float32),
            pltpu.SemaphoreType.DMA,
            pltpu.SemaphoreType.DMA,
            pltpu.SemaphoreType.DMA,
            pltpu.SemaphoreType.DMA,
        ],
        compiler_params=pltpu.CompilerParams(needs_layout_passes=False),
    )
    out = run(table_flat, idx)
    return out.reshape(*orig_shape[:-1], orig_shape[-1] * NUM_FUNCS)


# speedup vs baseline: 1.0177x; 1.0177x over previous
import jax
import jax.numpy as jnp
from jax import lax
from jax.experimental import pallas as pl
from jax.experimental.pallas import tpu as pltpu
from jax.experimental.pallas import tpu_sc as plsc

NUM_FUNCS = 32
TABLE_ROWS = 256
LANES = 16

_ROWS = 16384
_COLS = 100
_OUT_COLS = _COLS * NUM_FUNCS
_NW = 32
_ROWS_W = _ROWS // _NW
_CROWS = 16
_CHUNK = _CROWS * _COLS
_NCHUNK = _ROWS_W // _CROWS
_NBUF = 2


def _rbf_kernel(table_hbm, idx_hbm, out_hbm, table_v, idx_v0, idx_v1,
                out_v0, out_v1, si0, si1, so0, so1):
    wid = lax.axis_index("s") * 2 + lax.axis_index("c")
    wrow = wid * _ROWS_W
    idx_v = [idx_v0, idx_v1]
    out_v = [out_v0, out_v1]
    si = [si0, si1]
    so = [so0, so1]

    pltpu.sync_copy(table_hbm, table_v)

    lane = lax.iota(jnp.int32, LANES)

    def in_copy(b, g):
        r0 = wrow + g * _CROWS
        return pltpu.make_async_copy(
            idx_hbm.at[pl.ds(r0, _CROWS), :], idx_v[b], si[b])

    def out_copy(b, g):
        r0 = wrow + g * _CROWS
        return pltpu.make_async_copy(
            out_v[b], out_hbm.at[pl.ds(r0, _CROWS), :], so[b])

    for b in range(_NBUF):
        in_copy(b, b).start()

    def outer_body(o, carry):
        for b in range(_NBUF):
            g = o * _NBUF + b
            in_copy(b, g).wait()

            @pl.when(o > 0)
            def _wait_out():
                out_copy(b, g - _NBUF).wait()

            @plsc.parallel_loop(0, _CHUNK // (4 * LANES))
            def group_body(i):
                quads = []
                for h in range(4):
                    e = i * (4 * LANES) + h * LANES + lane
                    ri = (e * 5243) >> 19
                    c = e - ri * _COLS
                    r = plsc.load_gather(idx_v[b], [ri, c])
                    r = jnp.minimum(jnp.maximum(r, 0), TABLE_ROWS - 1)
                    quads.append((ri, c * NUM_FUNCS, r * NUM_FUNCS))

                @plsc.parallel_loop(0, NUM_FUNCS, unroll=4)
                def kbody(k):
                    jv = (lane + k) & (NUM_FUNCS - 1)
                    for ri, c32, r32 in quads:
                        vals = plsc.load_gather(table_v, [r32 + jv])
                        plsc.store_scatter(out_v[b], [ri, c32 + jv], vals)

            out_copy(b, g).start()

            @pl.when(o < _NCHUNK // _NBUF - 1)
            def _prefetch():
                in_copy(b, g + _NBUF).start()

        return carry

    lax.fori_loop(0, _NCHUNK // _NBUF, outer_body, 0)
    for b in range(_NBUF):
        out_copy(b, _NCHUNK - _NBUF + b).wait()


def kernel(tensor, int_to_feat_matrix, extra_embeddings):
    orig_shape = tensor.shape
    table = jnp.concatenate([int_to_feat_matrix, extra_embeddings], axis=0)
    table_flat = table.reshape(-1)
    idx = tensor.astype(jnp.int32)

    mesh = plsc.VectorSubcoreMesh(core_axis_name="c", subcore_axis_name="s")
    run = pl.kernel(
        _rbf_kernel,
        mesh=mesh,
        out_type=jax.ShapeDtypeStruct((_ROWS, _OUT_COLS), jnp.float32),
        scratch_types=[
            pltpu.VMEM((TABLE_ROWS * NUM_FUNCS,), jnp.float32),
            pltpu.VMEM((_CROWS, _COLS), jnp.int32),
            pltpu.VMEM((_CROWS, _COLS), jnp.int32),
            pltpu.VMEM((_CROWS, _OUT_COLS), jnp.float32),
            pltpu.VMEM((_CROWS, _OUT_COLS), jnp.float32),
            pltpu.SemaphoreType.DMA,
            pltpu.SemaphoreType.DMA,
            pltpu.SemaphoreType.DMA,
            pltpu.SemaphoreType.DMA,
        ],
        compiler_params=pltpu.CompilerParams(needs_layout_passes=False),
    )
    out = run(table_flat, idx)
    return out.reshape(*orig_shape[:-1], orig_shape[-1] * NUM_FUNCS)
